# Initial kernel scaffold; baseline (speedup 1.0000x reference)
import jax
import jax.numpy as jnp
from jax.experimental import pallas as pl


def kernel(indexseq, table):
    raise NotImplementedError("write your pallas kernel here")


# speedup vs baseline: 1.5846x; 1.5846x over previous
import functools

import jax
import jax.numpy as jnp
from jax import lax
from jax.experimental import pallas as pl
from jax.experimental.pallas import tpu as pltpu
from jax.experimental.pallas import tpu_sc as plsc

VOCAB = 1000000
EMBDIM = 32
B = 4096
L = 200

_NUM_WORKERS = 32
_TOTAL = B * L
_PER_W = _TOTAL // _NUM_WORKERS
_CHUNK = 1280
_NCH = _PER_W // _CHUNK

_mesh = plsc.VectorSubcoreMesh(core_axis_name="c", subcore_axis_name="s")


@functools.partial(
    pl.kernel,
    out_type=jax.ShapeDtypeStruct((_TOTAL, EMBDIM), jnp.float32),
    mesh=_mesh,
    compiler_params=pltpu.CompilerParams(use_tc_tiling_on_sc=False),
    scratch_types=[
        pltpu.VMEM((_PER_W,), jnp.int32),
        pltpu.VMEM((_CHUNK, EMBDIM), jnp.float32),
        pltpu.SemaphoreType.DMA,
    ],
)
def _emb_gather(idx_hbm, table_hbm, out_hbm, idx_v, rows_v, sem):
  wid = lax.axis_index("s") * 2 + lax.axis_index("c")
  base = wid * _PER_W
  pltpu.sync_copy(idx_hbm.at[pl.ds(base, _PER_W)], idx_v)

  def body(_, off):
    off = pl.multiple_of(off, 8)
    pltpu.async_copy(
        table_hbm.at[idx_v.at[pl.ds(off, _CHUNK)]], rows_v, sem
    ).wait()
    pltpu.sync_copy(rows_v, out_hbm.at[pl.ds(base + off, _CHUNK)])
    return off + jnp.int32(_CHUNK)

  lax.fori_loop(0, _NCH, body, jnp.int32(0), unroll=False)


def kernel(indexseq, table):
  idx = jnp.asarray(indexseq, jnp.int32).T.reshape(_TOTAL)
  out = _emb_gather(idx, table)
  return out.reshape(L, B, EMBDIM)


# speedup vs baseline: 1.6056x; 1.0132x over previous
import functools

import jax
import jax.numpy as jnp
from jax import lax
from jax.experimental import pallas as pl
from jax.experimental.pallas import tpu as pltpu
from jax.experimental.pallas import tpu_sc as plsc

VOCAB = 1000000
EMBDIM = 32
B = 4096
L = 200

_NUM_WORKERS = 32
_TOTAL = B * L
_PER_W = _TOTAL // _NUM_WORKERS
_CHUNK = 1280
_NCH = _PER_W // _CHUNK

_mesh = plsc.VectorSubcoreMesh(core_axis_name="c", subcore_axis_name="s")


_NBUF = 2


@functools.partial(
    pl.kernel,
    out_type=jax.ShapeDtypeStruct((_TOTAL, EMBDIM), jnp.float32),
    mesh=_mesh,
    compiler_params=pltpu.CompilerParams(use_tc_tiling_on_sc=False),
    scratch_types=[
        pltpu.VMEM((_PER_W,), jnp.int32),
        pltpu.VMEM((_NBUF, _CHUNK, EMBDIM), jnp.float32),
        pltpu.SemaphoreType.DMA((_NBUF,)),
        pltpu.SemaphoreType.DMA((_NBUF,)),
    ],
)
def _emb_gather(idx_hbm, table_hbm, out_hbm, idx_v, rows_v, gsem, osem):
  wid = lax.axis_index("s") * 2 + lax.axis_index("c")
  base = wid * _PER_W
  pltpu.sync_copy(idx_hbm.at[pl.ds(base, _PER_W)], idx_v)

  def gather(j, b):
    pltpu.async_copy(
        table_hbm.at[idx_v.at[pl.ds(j * _CHUNK, _CHUNK)]],
        rows_v.at[jnp.int32(b)],
        gsem.at[jnp.int32(b)],
    )

  def out_copy(j, b):
    return pltpu.make_async_copy(
        rows_v.at[jnp.int32(b)], out_hbm.at[pl.ds(base + j * _CHUNK, _CHUNK)], osem.at[jnp.int32(b)]
    )

  for j in range(_NCH):
    b = j % _NBUF
    if j >= _NBUF:
      out_copy(j - _NBUF, b).wait()
    gather(j, b)
    jj = j - (_NBUF - 1)
    if jj >= 0:
      bb = jj % _NBUF
      pltpu.make_async_copy(
          table_hbm.at[idx_v.at[pl.ds(jj * _CHUNK, _CHUNK)]],
          rows_v.at[jnp.int32(bb)],
          gsem.at[jnp.int32(bb)],
      ).wait()
      out_copy(jj, bb).start()
  for jj in range(_NCH - (_NBUF - 1), _NCH):
    bb = jj % _NBUF
    pltpu.make_async_copy(
        table_hbm.at[idx_v.at[pl.ds(jj * _CHUNK, _CHUNK)]],
        rows_v.at[jnp.int32(bb)],
        gsem.at[jnp.int32(bb)],
    ).wait()
    out_copy(jj, bb).start()
  for jj in range(_NCH - _NBUF, _NCH):
    out_copy(jj, jj % _NBUF).wait()


def kernel(indexseq, table):
  idx = jnp.asarray(indexseq, jnp.int32).T.reshape(_TOTAL)
  out = _emb_gather(idx, table)
  return out.reshape(L, B, EMBDIM)
